# baseline (speedup 1.0000x reference)
import jax
import jax.numpy as jnp
from jax.experimental import pallas as pl
from jax.experimental.pallas import tpu as pltpu

_NCH = 4


def _fused_vo_body(x_ref, bv_ref, bo_ref, wv_hbm, wo_hbm, y_ref,
                   wv_buf, wo_buf, vsem, osem):
    d_wm = wv_hbm.shape[1]
    ch = d_wm // _NCH

    def start(c):
        slot = c % 2
        pltpu.make_async_copy(
            wv_hbm.at[:, pl.ds(c * ch, ch)], wv_buf.at[slot], vsem.at[slot]
        ).start()
        pltpu.make_async_copy(
            wo_hbm.at[pl.ds(c * ch, ch), :], wo_buf.at[slot], osem.at[slot]
        ).start()

    def wait(c):
        slot = c % 2
        pltpu.make_async_copy(
            wv_hbm.at[:, pl.ds(c * ch, ch)], wv_buf.at[slot], vsem.at[slot]
        ).wait()
        pltpu.make_async_copy(
            wo_hbm.at[pl.ds(c * ch, ch), :], wo_buf.at[slot], osem.at[slot]
        ).wait()

    start(0)
    for c in range(_NCH):
        if c + 1 < _NCH:
            start(c + 1)
        wait(c)
        slot = c % 2
        v = jnp.dot(x_ref[...], wv_buf[slot],
                    preferred_element_type=jnp.float32)
        v = v + bv_ref[:, c * ch:(c + 1) * ch]
        part = jnp.dot(v, wo_buf[slot], preferred_element_type=jnp.float32)
        if c == 0:
            y_ref[...] = part + bo_ref[...]
        else:
            y_ref[...] += part


def kernel(x, reset_mask, Wq, bq, Wk, bk, Wv, bv, Wo, bo):
    del reset_mask, Wq, bq, Wk, bk
    bs, d = x.shape
    d_wm = Wv.shape[1]
    ch = d_wm // _NCH
    return pl.pallas_call(
        _fused_vo_body,
        in_specs=[
            pl.BlockSpec((bs, d), lambda: (0, 0)),
            pl.BlockSpec((1, d_wm), lambda: (0, 0)),
            pl.BlockSpec((1, d), lambda: (0, 0)),
            pl.BlockSpec(memory_space=pl.ANY),
            pl.BlockSpec(memory_space=pl.ANY),
        ],
        out_specs=pl.BlockSpec((bs, d), lambda: (0, 0)),
        out_shape=jax.ShapeDtypeStruct((bs, d), jnp.float32),
        scratch_shapes=[
            pltpu.VMEM((2, d, ch), jnp.float32),
            pltpu.VMEM((2, ch, d), jnp.float32),
            pltpu.SemaphoreType.DMA((2,)),
            pltpu.SemaphoreType.DMA((2,)),
        ],
    )(x, bv.reshape(1, -1), bo.reshape(1, -1), Wv, Wo)


# speedup vs baseline: 1.0436x; 1.0436x over previous
import jax
import jax.numpy as jnp
from jax.experimental import pallas as pl
from jax.experimental.pallas import tpu as pltpu

_NCH = 4


def _fused_vo_body(x_ref, bv_ref, bo_ref, wv_hbm, wo_hbm, y_ref,
                   wv_buf, wo_buf, v_acc, vsem, osem):
    d = wv_hbm.shape[0]
    ch = d // _NCH

    def cp_v(c):
        return pltpu.make_async_copy(
            wv_hbm.at[pl.ds(c * ch, ch), :], wv_buf.at[c], vsem.at[c])

    def cp_o(c):
        return pltpu.make_async_copy(
            wo_hbm.at[pl.ds(c * ch, ch), :], wo_buf.at[c], osem.at[c])

    for c in range(_NCH):
        cp_v(c).start()
    for c in range(_NCH):
        cp_o(c).start()

    for c in range(_NCH):
        cp_v(c).wait()
        part = jnp.dot(x_ref[:, c * ch:(c + 1) * ch], wv_buf[c],
                       preferred_element_type=jnp.float32)
        if c == 0:
            v_acc[...] = part + bv_ref[...]
        else:
            v_acc[...] += part

    for c in range(_NCH):
        cp_o(c).wait()
        part = jnp.dot(v_acc[:, c * ch:(c + 1) * ch], wo_buf[c],
                       preferred_element_type=jnp.float32)
        if c == 0:
            y_ref[...] = part + bo_ref[...]
        else:
            y_ref[...] += part


def kernel(x, reset_mask, Wq, bq, Wk, bk, Wv, bv, Wo, bo):
    del reset_mask, Wq, bq, Wk, bk
    bs, d = x.shape
    d_wm = Wv.shape[1]
    ch = d // _NCH
    return pl.pallas_call(
        _fused_vo_body,
        in_specs=[
            pl.BlockSpec((bs, d), lambda: (0, 0)),
            pl.BlockSpec((1, d_wm), lambda: (0, 0)),
            pl.BlockSpec((1, d), lambda: (0, 0)),
            pl.BlockSpec(memory_space=pl.ANY),
            pl.BlockSpec(memory_space=pl.ANY),
        ],
        out_specs=pl.BlockSpec((bs, d), lambda: (0, 0)),
        out_shape=jax.ShapeDtypeStruct((bs, d), jnp.float32),
        scratch_shapes=[
            pltpu.VMEM((_NCH, ch, d_wm), jnp.float32),
            pltpu.VMEM((_NCH, ch, d), jnp.float32),
            pltpu.VMEM((bs, d_wm), jnp.float32),
            pltpu.SemaphoreType.DMA((_NCH,)),
            pltpu.SemaphoreType.DMA((_NCH,)),
        ],
    )(x, bv.reshape(1, -1), bo.reshape(1, -1), Wv, Wo)
